# baseline (speedup 1.0000x reference)
import functools

import numpy as np
import jax
import jax.numpy as jnp
from jax import lax
from jax.experimental import pallas as pl
from jax.experimental.pallas import tpu as pltpu
from jax.experimental.pallas import tpu_sc as plsc

EMB = 16
CHUNK = 250
CHUNKI = 256
G = 8
NTILES = 32
NSUB = 16
BN = 1024


def _pads(n_nodes, n_edges):
    npad = ((n_nodes + 1 + BN - 1) // BN) * BN
    kpt = ((-(-n_edges // (NTILES * CHUNK)) + G - 1) // G) * G
    return npad, kpt


def _sc_mesh():
    return plsc.VectorSubcoreMesh(core_axis_name="c", subcore_axis_name="s",
                                  num_cores=2, num_subcores=NSUB)


def _sc_edge_pass(ea_pad, dst_rows, zeros_hbm, npad, kpt):
    rps = npad // NSUB

    @functools.partial(
        pl.kernel,
        out_type=(
            jax.ShapeDtypeStruct((2, npad, EMB), jnp.float32),
            jax.ShapeDtypeStruct((2, npad, EMB), jnp.float32),
        ),
        mesh=_sc_mesh(),
        compiler_params=pltpu.CompilerParams(use_tc_tiling_on_sc=False),
        scratch_types=[
            pltpu.VMEM((kpt, CHUNKI), jnp.int32),
            pltpu.VMEM((G, CHUNKI, EMB), jnp.float32),
            pltpu.VMEM((CHUNKI, EMB), jnp.float32),
            pltpu.VMEM_SHARED((npad, EMB), jnp.float32),
            pltpu.VMEM_SHARED((npad, EMB), jnp.float32),
            pltpu.SemaphoreType.DMA((G,)),
            pltpu.SemaphoreType.DMA,
        ],
    )
    def k(ea_hbm, dsti_hbm, zo_hbm, out_ea, out_deg,
          dst_v, rows_v, ones_v, acc_sh, deg_sh, gsem, lsem):
        c = lax.axis_index("c")
        s = lax.axis_index("s")
        w = c * np.int32(NSUB) + s

        def _ob(i, carry):
            ones_v[i, :] = jnp.ones((EMB,), jnp.float32)
            return carry
        lax.fori_loop(np.int32(0), np.int32(CHUNKI), _ob, np.int32(0))

        base = s * np.int32(rps)
        pltpu.sync_copy(zo_hbm, acc_sh.at[pl.ds(base, rps)])
        pltpu.sync_copy(zo_hbm, deg_sh.at[pl.ds(base, rps)])
        pltpu.async_copy(dsti_hbm.at[pl.ds(w * np.int32(kpt), kpt)], dst_v,
                         lsem).wait()
        plsc.subcore_barrier()

        ebase = w * np.int32(kpt * CHUNK)

        def _gb(g, carry):
            cb = g * np.int32(G)
            handles = []
            for i in range(G):
                handles.append(pltpu.async_copy(
                    ea_hbm.at[pl.ds(ebase + (cb + np.int32(i))
                                    * np.int32(CHUNK), CHUNK)],
                    rows_v.at[i].at[pl.ds(0, CHUNK)], gsem.at[i]))
            for i in range(G):
                handles[i].wait()
                pltpu.sync_copy(rows_v.at[i],
                                acc_sh.at[dst_v.at[cb + np.int32(i)]],
                                add=True)
                pltpu.sync_copy(ones_v,
                                deg_sh.at[dst_v.at[cb + np.int32(i)]],
                                add=True)
            return carry
        lax.fori_loop(np.int32(0), np.int32(kpt // G), _gb, np.int32(0))

        plsc.subcore_barrier()
        pltpu.sync_copy(acc_sh.at[pl.ds(base, rps)],
                        out_ea.at[c].at[pl.ds(base, rps)])
        pltpu.sync_copy(deg_sh.at[pl.ds(base, rps)],
                        out_deg.at[c].at[pl.ds(base, rps)])

    with jax.enable_x64(False):
        return k(ea_pad, dst_rows, zeros_hbm)


def _sc_spmm(m_pad, src_rows, dst_rows, zeros_hbm, npad, kpt):
    rps = npad // NSUB

    @functools.partial(
        pl.kernel,
        out_type=jax.ShapeDtypeStruct((2, npad, EMB), jnp.float32),
        mesh=_sc_mesh(),
        compiler_params=pltpu.CompilerParams(use_tc_tiling_on_sc=False),
        scratch_types=[
            pltpu.VMEM((kpt, CHUNKI), jnp.int32),
            pltpu.VMEM((kpt, CHUNKI), jnp.int32),
            pltpu.VMEM((G, CHUNKI, EMB), jnp.float32),
            pltpu.VMEM_SHARED((npad, EMB), jnp.float32),
            pltpu.SemaphoreType.DMA((G,)),
            pltpu.SemaphoreType.DMA,
        ],
    )
    def k(m_hbm, srci_hbm, dsti_hbm, zo_hbm, out,
          src_v, dst_v, rows_v, acc_sh, gsem, lsem):
        c = lax.axis_index("c")
        s = lax.axis_index("s")
        w = c * np.int32(NSUB) + s

        base = s * np.int32(rps)
        pltpu.sync_copy(zo_hbm, acc_sh.at[pl.ds(base, rps)])
        pltpu.async_copy(srci_hbm.at[pl.ds(w * np.int32(kpt), kpt)], src_v,
                         lsem).wait()
        pltpu.async_copy(dsti_hbm.at[pl.ds(w * np.int32(kpt), kpt)], dst_v,
                         lsem).wait()
        plsc.subcore_barrier()

        def _gb(g, carry):
            cb = g * np.int32(G)
            handles = []
            for i in range(G):
                handles.append(pltpu.async_copy(
                    m_hbm.at[src_v.at[cb + np.int32(i)]], rows_v.at[i],
                    gsem.at[i]))
            for i in range(G):
                handles[i].wait()
                pltpu.sync_copy(rows_v.at[i],
                                acc_sh.at[dst_v.at[cb + np.int32(i)]],
                                add=True)
            return carry
        lax.fori_loop(np.int32(0), np.int32(kpt // G), _gb, np.int32(0))

        plsc.subcore_barrier()
        pltpu.sync_copy(acc_sh.at[pl.ds(base, rps)],
                        out.at[c].at[pl.ds(base, rps)])

    with jax.enable_x64(False):
        return k(m_pad, src_rows, dst_rows, zeros_hbm)


def _recur(mu, w1_ref, b1_ref, w2_ref, b2_ref):
    m = jax.nn.relu(jnp.dot(mu, w1_ref[...],
                            preferred_element_type=jnp.float32) + b1_ref[...])
    return jax.nn.relu(jnp.dot(m, w2_ref[...],
                               preferred_element_type=jnp.float32) + b2_ref[...])


def _tc_combine0(acc_ea, acc_deg, xp, mean, std, W_node, b_node,
                 W_edge, b_edge, W_r1, b_r1, W_r2, b_r2, npad):
    df = xp.shape[1]

    def body(ea_ref, dg_ref, x_ref, mn_ref, sd_ref, wn_ref, bn_ref,
             we_ref, be_ref, w1_ref, b1_ref, w2_ref, b2_ref,
             hn_ref, agg_ref, m_ref):
        xb = (x_ref[...] - mn_ref[...]) / sd_ref[...]
        h_node = jnp.dot(xb, wn_ref[...],
                         preferred_element_type=jnp.float32) + bn_ref[...]
        ea = ea_ref[0] + ea_ref[1]
        deg = (dg_ref[0] + dg_ref[1])[:, 0:1]
        agg_e = jnp.dot(ea, we_ref[...],
                        preferred_element_type=jnp.float32) + deg * be_ref[...]
        m0 = _recur(jnp.zeros((1, EMB), jnp.float32), w1_ref, b1_ref,
                    w2_ref, b2_ref)
        mu1 = jax.nn.relu(h_node + agg_e + deg * m0)
        hn_ref[...] = h_node
        agg_ref[...] = agg_e
        m_ref[...] = _recur(mu1, w1_ref, b1_ref, w2_ref, b2_ref)

    w16 = pl.BlockSpec((EMB, EMB), lambda i: (0, 0))
    b16 = pl.BlockSpec((1, EMB), lambda i: (0, 0))
    return pl.pallas_call(
        body,
        grid=(npad // BN,),
        in_specs=[
            pl.BlockSpec((2, BN, EMB), lambda i: (0, i, 0)),
            pl.BlockSpec((2, BN, EMB), lambda i: (0, i, 0)),
            pl.BlockSpec((BN, df), lambda i: (i, 0)),
            pl.BlockSpec((1, df), lambda i: (0, 0)),
            pl.BlockSpec((1, df), lambda i: (0, 0)),
            pl.BlockSpec((df, EMB), lambda i: (0, 0)),
            b16, w16, b16, w16, b16, w16, b16,
        ],
        out_specs=[pl.BlockSpec((BN, EMB), lambda i: (i, 0))] * 3,
        out_shape=[jax.ShapeDtypeStruct((npad, EMB), jnp.float32)] * 3,
    )(acc_ea, acc_deg, xp, mean.reshape(1, df), std.reshape(1, df),
      W_node, b_node.reshape(1, EMB), W_edge, b_edge.reshape(1, EMB),
      W_r1, b_r1.reshape(1, EMB), W_r2, b_r2.reshape(1, EMB))


def _tc_step(p, h_node, agg_e, W_r1, b_r1, W_r2, b_r2, npad):
    def body(p_ref, hn_ref, ag_ref, w1_ref, b1_ref, w2_ref, b2_ref, m_ref):
        mu = jax.nn.relu(hn_ref[...] + p_ref[0] + p_ref[1] + ag_ref[...])
        m_ref[...] = _recur(mu, w1_ref, b1_ref, w2_ref, b2_ref)

    w16 = pl.BlockSpec((EMB, EMB), lambda i: (0, 0))
    b16 = pl.BlockSpec((1, EMB), lambda i: (0, 0))
    return pl.pallas_call(
        body,
        grid=(npad // BN,),
        in_specs=[
            pl.BlockSpec((2, BN, EMB), lambda i: (0, i, 0)),
            pl.BlockSpec((BN, EMB), lambda i: (i, 0)),
            pl.BlockSpec((BN, EMB), lambda i: (i, 0)),
            w16, b16, w16, b16,
        ],
        out_specs=pl.BlockSpec((BN, EMB), lambda i: (i, 0)),
        out_shape=jax.ShapeDtypeStruct((npad, EMB), jnp.float32),
    )(p, h_node, agg_e, W_r1, b_r1.reshape(1, EMB), W_r2,
      b_r2.reshape(1, EMB))


def _tc_classify(p, h_node, agg_e, W_c1, b_c1, W_c2, b_c2, Wl, bl, npad):
    nc = Wl.shape[1]

    def body(p_ref, hn_ref, ag_ref, w1_ref, b1_ref, w2_ref, b2_ref,
             wl_ref, bl_ref, o_ref):
        mu = jax.nn.relu(hn_ref[...] + p_ref[0] + p_ref[1] + ag_ref[...])
        h = jax.nn.relu(jnp.dot(mu, w1_ref[...],
                                preferred_element_type=jnp.float32)
                        + b1_ref[...])
        h = jax.nn.relu(jnp.dot(h, w2_ref[...],
                                preferred_element_type=jnp.float32)
                        + b2_ref[...])
        lg = jnp.dot(h, wl_ref[...],
                     preferred_element_type=jnp.float32) + bl_ref[...]
        lg = lg - jnp.max(lg, axis=-1, keepdims=True)
        o_ref[...] = lg - jnp.log(jnp.sum(jnp.exp(lg), axis=-1,
                                          keepdims=True))

    w16 = pl.BlockSpec((EMB, EMB), lambda i: (0, 0))
    b16 = pl.BlockSpec((1, EMB), lambda i: (0, 0))
    return pl.pallas_call(
        body,
        grid=(npad // BN,),
        in_specs=[
            pl.BlockSpec((2, BN, EMB), lambda i: (0, i, 0)),
            pl.BlockSpec((BN, EMB), lambda i: (i, 0)),
            pl.BlockSpec((BN, EMB), lambda i: (i, 0)),
            w16, b16, w16, b16,
            pl.BlockSpec((EMB, nc), lambda i: (0, 0)),
            pl.BlockSpec((1, nc), lambda i: (0, 0)),
        ],
        out_specs=pl.BlockSpec((BN, nc), lambda i: (i, 0)),
        out_shape=jax.ShapeDtypeStruct((npad, nc), jnp.float32),
    )(p, h_node, agg_e, W_c1, b_c1.reshape(1, EMB), W_c2,
      b_c2.reshape(1, EMB), Wl, bl.reshape(1, nc))


def kernel(x, edge_index, edge_attr, mean, std,
           W_node, b_node, W_edge, b_edge,
           W_r1, b_r1, W_r2, b_r2,
           W_c1, b_c1, W_c2, b_c2,
           W_log, b_log):
    with jax.enable_x64(False):
        n, df = x.shape
        e = edge_attr.shape[0]
        npad, kpt = _pads(n, e)
        epad = NTILES * kpt * CHUNK

        f32 = jnp.float32
        src = edge_index[0].astype(jnp.int32)
        dst = edge_index[1].astype(jnp.int32)
        pad_e = epad - e
        if pad_e:
            src = jnp.concatenate([src, jnp.zeros((pad_e,), jnp.int32)])
            dst = jnp.concatenate([dst, jnp.full((pad_e,), npad - 1,
                                                 jnp.int32)])
        pad_i = CHUNKI - CHUNK
        src_rows = jnp.pad(src.reshape(NTILES * kpt, CHUNK),
                           ((0, 0), (0, pad_i)))
        dst_rows = jnp.pad(dst.reshape(NTILES * kpt, CHUNK),
                           ((0, 0), (0, pad_i)),
                           constant_values=npad - 1)
        ea_pad = edge_attr if edge_attr.dtype == f32 else edge_attr.astype(f32)
        if pad_e:
            ea_pad = jnp.concatenate([ea_pad, jnp.zeros((pad_e, EMB), f32)])
        zeros_hbm = jnp.zeros((npad // NSUB, EMB), f32)
        xp = jnp.pad(x.astype(f32), ((0, npad - n), (0, 0)))

        acc_ea, acc_deg = _sc_edge_pass(ea_pad, dst_rows, zeros_hbm,
                                        npad, kpt)
        h_node, agg_e, m = _tc_combine0(
            acc_ea, acc_deg, xp, mean.astype(f32), std.astype(f32),
            W_node.astype(f32), b_node.astype(f32), W_edge.astype(f32),
            b_edge.astype(f32), W_r1.astype(f32), b_r1.astype(f32),
            W_r2.astype(f32), b_r2.astype(f32), npad)
        for _ in range(2):
            p = _sc_spmm(m, src_rows, dst_rows, zeros_hbm, npad, kpt)
            m = _tc_step(p, h_node, agg_e, W_r1.astype(f32),
                         b_r1.astype(f32),
                         W_r2.astype(f32), b_r2.astype(f32), npad)
        p = _sc_spmm(m, src_rows, dst_rows, zeros_hbm, npad, kpt)
        out32 = _tc_classify(p, h_node, agg_e, W_c1.astype(f32),
                             b_c1.astype(f32), W_c2.astype(f32),
                             b_c2.astype(f32), W_log.astype(f32),
                             b_log.astype(f32), npad)
    return out32[:n].astype(jnp.float64)


# speedup vs baseline: 1.2022x; 1.2022x over previous
import functools

import numpy as np
import jax
import jax.numpy as jnp
from jax import lax
from jax.experimental import pallas as pl
from jax.experimental.pallas import tpu as pltpu
from jax.experimental.pallas import tpu_sc as plsc

EMB = 16
CHUNK = 250
G = 8
NTILES = 32
NSUB = 16
BN = 1024


def _pads(n_nodes, n_edges):
    npad = ((n_nodes + 1 + BN - 1) // BN) * BN
    kpt = ((-(-n_edges // (NTILES * CHUNK)) + G - 1) // G) * G
    return npad, kpt


def _sc_mesh():
    return plsc.VectorSubcoreMesh(core_axis_name="c", subcore_axis_name="s",
                                  num_cores=2, num_subcores=NSUB)


def _sc_edge_pass(ea_pad, dst_rows, zeros_hbm, npad, kpt):
    rps = npad // NSUB

    @functools.partial(
        pl.kernel,
        out_type=(
            jax.ShapeDtypeStruct((2, npad, EMB), jnp.float32),
            jax.ShapeDtypeStruct((2, npad, EMB), jnp.float32),
        ),
        mesh=_sc_mesh(),
        compiler_params=pltpu.CompilerParams(use_tc_tiling_on_sc=False),
        scratch_types=[
            pltpu.VMEM((kpt, CHUNK), jnp.int32),
            pltpu.VMEM((G, CHUNK, EMB), jnp.float32),
            pltpu.VMEM((CHUNK, EMB), jnp.float32),
            pltpu.VMEM_SHARED((npad, EMB), jnp.float32),
            pltpu.VMEM_SHARED((npad, EMB), jnp.float32),
            pltpu.SemaphoreType.DMA((G,)),
            pltpu.SemaphoreType.DMA,
        ],
    )
    def k(ea_hbm, dsti_hbm, zo_hbm, out_ea, out_deg,
          dst_v, rows_v, ones_v, acc_sh, deg_sh, gsem, lsem):
        c = lax.axis_index("c")
        s = lax.axis_index("s")
        w = c * np.int32(NSUB) + s

        def _ob(i, carry):
            ones_v[i, :] = jnp.ones((EMB,), jnp.float32)
            return carry
        lax.fori_loop(np.int32(0), np.int32(CHUNK), _ob, np.int32(0))

        base = s * np.int32(rps)
        pltpu.sync_copy(zo_hbm, acc_sh.at[pl.ds(base, rps)])
        pltpu.sync_copy(zo_hbm, deg_sh.at[pl.ds(base, rps)])
        pltpu.async_copy(dsti_hbm.at[pl.ds(w * np.int32(kpt), kpt)], dst_v,
                         lsem).wait()
        plsc.subcore_barrier()

        ebase = w * np.int32(kpt * CHUNK)

        def _gb(g, carry):
            cb = g * np.int32(G)
            handles = []
            for i in range(G):
                handles.append(pltpu.async_copy(
                    ea_hbm.at[pl.ds(ebase + (cb + np.int32(i))
                                    * np.int32(CHUNK), CHUNK)],
                    rows_v.at[i], gsem.at[i]))
            for i in range(G):
                handles[i].wait()
                pltpu.sync_copy(rows_v.at[i],
                                acc_sh.at[dst_v.at[cb + np.int32(i)]],
                                add=True)
                pltpu.sync_copy(ones_v,
                                deg_sh.at[dst_v.at[cb + np.int32(i)]],
                                add=True)
            return carry
        lax.fori_loop(np.int32(0), np.int32(kpt // G), _gb, np.int32(0))

        plsc.subcore_barrier()
        pltpu.sync_copy(acc_sh.at[pl.ds(base, rps)],
                        out_ea.at[c].at[pl.ds(base, rps)])
        pltpu.sync_copy(deg_sh.at[pl.ds(base, rps)],
                        out_deg.at[c].at[pl.ds(base, rps)])

    with jax.enable_x64(False):
        return k(ea_pad, dst_rows, zeros_hbm)


def _sc_spmm(m_pad, src_rows, dst_rows, zeros_hbm, npad, kpt):
    rps = npad // NSUB

    @functools.partial(
        pl.kernel,
        out_type=jax.ShapeDtypeStruct((2, npad, EMB), jnp.float32),
        mesh=_sc_mesh(),
        compiler_params=pltpu.CompilerParams(use_tc_tiling_on_sc=False),
        scratch_types=[
            pltpu.VMEM((kpt, CHUNK), jnp.int32),
            pltpu.VMEM((kpt, CHUNK), jnp.int32),
            pltpu.VMEM((G, CHUNK, EMB), jnp.float32),
            pltpu.VMEM_SHARED((npad, EMB), jnp.float32),
            pltpu.SemaphoreType.DMA((G,)),
            pltpu.SemaphoreType.DMA,
        ],
    )
    def k(m_hbm, srci_hbm, dsti_hbm, zo_hbm, out,
          src_v, dst_v, rows_v, acc_sh, gsem, lsem):
        c = lax.axis_index("c")
        s = lax.axis_index("s")
        w = c * np.int32(NSUB) + s

        base = s * np.int32(rps)
        pltpu.sync_copy(zo_hbm, acc_sh.at[pl.ds(base, rps)])
        pltpu.async_copy(srci_hbm.at[pl.ds(w * np.int32(kpt), kpt)], src_v,
                         lsem).wait()
        pltpu.async_copy(dsti_hbm.at[pl.ds(w * np.int32(kpt), kpt)], dst_v,
                         lsem).wait()
        plsc.subcore_barrier()

        def _gb(g, carry):
            cb = g * np.int32(G)
            handles = []
            for i in range(G):
                handles.append(pltpu.async_copy(
                    m_hbm.at[src_v.at[cb + np.int32(i)]], rows_v.at[i],
                    gsem.at[i]))
            for i in range(G):
                handles[i].wait()
                pltpu.sync_copy(rows_v.at[i],
                                acc_sh.at[dst_v.at[cb + np.int32(i)]],
                                add=True)
            return carry
        lax.fori_loop(np.int32(0), np.int32(kpt // G), _gb, np.int32(0))

        plsc.subcore_barrier()
        pltpu.sync_copy(acc_sh.at[pl.ds(base, rps)],
                        out.at[c].at[pl.ds(base, rps)])

    with jax.enable_x64(False):
        return k(m_pad, src_rows, dst_rows, zeros_hbm)


def _recur(mu, w1_ref, b1_ref, w2_ref, b2_ref):
    m = jax.nn.relu(jnp.dot(mu, w1_ref[...],
                            preferred_element_type=jnp.float32) + b1_ref[...])
    return jax.nn.relu(jnp.dot(m, w2_ref[...],
                               preferred_element_type=jnp.float32) + b2_ref[...])


def _tc_combine0(acc_ea, acc_deg, xp, mean, std, W_node, b_node,
                 W_edge, b_edge, W_r1, b_r1, W_r2, b_r2, npad):
    df = xp.shape[1]

    def body(ea_ref, dg_ref, x_ref, mn_ref, sd_ref, wn_ref, bn_ref,
             we_ref, be_ref, w1_ref, b1_ref, w2_ref, b2_ref,
             hn_ref, agg_ref, m_ref):
        xb = (x_ref[...] - mn_ref[...]) / sd_ref[...]
        h_node = jnp.dot(xb, wn_ref[...],
                         preferred_element_type=jnp.float32) + bn_ref[...]
        ea = ea_ref[0] + ea_ref[1]
        deg = (dg_ref[0] + dg_ref[1])[:, 0:1]
        agg_e = jnp.dot(ea, we_ref[...],
                        preferred_element_type=jnp.float32) + deg * be_ref[...]
        m0 = _recur(jnp.zeros((1, EMB), jnp.float32), w1_ref, b1_ref,
                    w2_ref, b2_ref)
        mu1 = jax.nn.relu(h_node + agg_e + deg * m0)
        hn_ref[...] = h_node
        agg_ref[...] = agg_e
        m_ref[...] = _recur(mu1, w1_ref, b1_ref, w2_ref, b2_ref)

    w16 = pl.BlockSpec((EMB, EMB), lambda i: (0, 0))
    b16 = pl.BlockSpec((1, EMB), lambda i: (0, 0))
    return pl.pallas_call(
        body,
        grid=(npad // BN,),
        in_specs=[
            pl.BlockSpec((2, BN, EMB), lambda i: (0, i, 0)),
            pl.BlockSpec((2, BN, EMB), lambda i: (0, i, 0)),
            pl.BlockSpec((BN, df), lambda i: (i, 0)),
            pl.BlockSpec((1, df), lambda i: (0, 0)),
            pl.BlockSpec((1, df), lambda i: (0, 0)),
            pl.BlockSpec((df, EMB), lambda i: (0, 0)),
            b16, w16, b16, w16, b16, w16, b16,
        ],
        out_specs=[pl.BlockSpec((BN, EMB), lambda i: (i, 0))] * 3,
        out_shape=[jax.ShapeDtypeStruct((npad, EMB), jnp.float32)] * 3,
    )(acc_ea, acc_deg, xp, mean.reshape(1, df), std.reshape(1, df),
      W_node, b_node.reshape(1, EMB), W_edge, b_edge.reshape(1, EMB),
      W_r1, b_r1.reshape(1, EMB), W_r2, b_r2.reshape(1, EMB))


def _tc_step(p, h_node, agg_e, W_r1, b_r1, W_r2, b_r2, npad):
    def body(p_ref, hn_ref, ag_ref, w1_ref, b1_ref, w2_ref, b2_ref, m_ref):
        mu = jax.nn.relu(hn_ref[...] + p_ref[0] + p_ref[1] + ag_ref[...])
        m_ref[...] = _recur(mu, w1_ref, b1_ref, w2_ref, b2_ref)

    w16 = pl.BlockSpec((EMB, EMB), lambda i: (0, 0))
    b16 = pl.BlockSpec((1, EMB), lambda i: (0, 0))
    return pl.pallas_call(
        body,
        grid=(npad // BN,),
        in_specs=[
            pl.BlockSpec((2, BN, EMB), lambda i: (0, i, 0)),
            pl.BlockSpec((BN, EMB), lambda i: (i, 0)),
            pl.BlockSpec((BN, EMB), lambda i: (i, 0)),
            w16, b16, w16, b16,
        ],
        out_specs=pl.BlockSpec((BN, EMB), lambda i: (i, 0)),
        out_shape=jax.ShapeDtypeStruct((npad, EMB), jnp.float32),
    )(p, h_node, agg_e, W_r1, b_r1.reshape(1, EMB), W_r2,
      b_r2.reshape(1, EMB))


def _tc_classify(p, h_node, agg_e, W_c1, b_c1, W_c2, b_c2, Wl, bl, npad):
    nc = Wl.shape[1]

    def body(p_ref, hn_ref, ag_ref, w1_ref, b1_ref, w2_ref, b2_ref,
             wl_ref, bl_ref, o_ref):
        mu = jax.nn.relu(hn_ref[...] + p_ref[0] + p_ref[1] + ag_ref[...])
        h = jax.nn.relu(jnp.dot(mu, w1_ref[...],
                                preferred_element_type=jnp.float32)
                        + b1_ref[...])
        h = jax.nn.relu(jnp.dot(h, w2_ref[...],
                                preferred_element_type=jnp.float32)
                        + b2_ref[...])
        lg = jnp.dot(h, wl_ref[...],
                     preferred_element_type=jnp.float32) + bl_ref[...]
        lg = lg - jnp.max(lg, axis=-1, keepdims=True)
        o_ref[...] = lg - jnp.log(jnp.sum(jnp.exp(lg), axis=-1,
                                          keepdims=True))

    w16 = pl.BlockSpec((EMB, EMB), lambda i: (0, 0))
    b16 = pl.BlockSpec((1, EMB), lambda i: (0, 0))
    return pl.pallas_call(
        body,
        grid=(npad // BN,),
        in_specs=[
            pl.BlockSpec((2, BN, EMB), lambda i: (0, i, 0)),
            pl.BlockSpec((BN, EMB), lambda i: (i, 0)),
            pl.BlockSpec((BN, EMB), lambda i: (i, 0)),
            w16, b16, w16, b16,
            pl.BlockSpec((EMB, nc), lambda i: (0, 0)),
            pl.BlockSpec((1, nc), lambda i: (0, 0)),
        ],
        out_specs=pl.BlockSpec((BN, nc), lambda i: (i, 0)),
        out_shape=jax.ShapeDtypeStruct((npad, nc), jnp.float32),
    )(p, h_node, agg_e, W_c1, b_c1.reshape(1, EMB), W_c2,
      b_c2.reshape(1, EMB), Wl, bl.reshape(1, nc))


def kernel(x, edge_index, edge_attr, mean, std,
           W_node, b_node, W_edge, b_edge,
           W_r1, b_r1, W_r2, b_r2,
           W_c1, b_c1, W_c2, b_c2,
           W_log, b_log):
    with jax.enable_x64(False):
        n, df = x.shape
        e = edge_attr.shape[0]
        npad, kpt = _pads(n, e)
        epad = NTILES * kpt * CHUNK

        f32 = jnp.float32
        src = edge_index[0].astype(jnp.int32)
        dst = edge_index[1].astype(jnp.int32)
        pad_e = epad - e
        if pad_e:
            src = jnp.concatenate([src, jnp.zeros((pad_e,), jnp.int32)])
            dst = jnp.concatenate([dst, jnp.full((pad_e,), npad - 1,
                                                 jnp.int32)])
        src_rows = src.reshape(NTILES * kpt, CHUNK)
        dst_rows = dst.reshape(NTILES * kpt, CHUNK)
        ea_pad = edge_attr if edge_attr.dtype == f32 else edge_attr.astype(f32)
        if pad_e:
            ea_pad = jnp.concatenate([ea_pad, jnp.zeros((pad_e, EMB), f32)])
        zeros_hbm = jnp.zeros((npad // NSUB, EMB), f32)
        xp = jnp.pad(x.astype(f32), ((0, npad - n), (0, 0)))

        acc_ea, acc_deg = _sc_edge_pass(ea_pad, dst_rows, zeros_hbm,
                                        npad, kpt)
        h_node, agg_e, m = _tc_combine0(
            acc_ea, acc_deg, xp, mean.astype(f32), std.astype(f32),
            W_node.astype(f32), b_node.astype(f32), W_edge.astype(f32),
            b_edge.astype(f32), W_r1.astype(f32), b_r1.astype(f32),
            W_r2.astype(f32), b_r2.astype(f32), npad)
        for _ in range(2):
            p = _sc_spmm(m, src_rows, dst_rows, zeros_hbm, npad, kpt)
            m = _tc_step(p, h_node, agg_e, W_r1.astype(f32),
                         b_r1.astype(f32),
                         W_r2.astype(f32), b_r2.astype(f32), npad)
        p = _sc_spmm(m, src_rows, dst_rows, zeros_hbm, npad, kpt)
        out32 = _tc_classify(p, h_node, agg_e, W_c1.astype(f32),
                             b_c1.astype(f32), W_c2.astype(f32),
                             b_c2.astype(f32), W_log.astype(f32),
                             b_log.astype(f32), npad)
    return out32[:n].astype(jnp.float64)
